# baseline (speedup 1.0000x reference)
import jax
import jax.numpy as jnp
from jax.experimental import pallas as pl


_TM = 2048
_PK = 128

_DN_T_RHS = (((1,), (1,)), ((), ()))


def _proj_body(emb_ref, smp_ref, w_ref, wp_ref, b_ref, out_ref):
    emb_dim = emb_ref.shape[1]
    acc = jax.lax.dot_general(emb_ref[...], w_ref[:, :emb_dim], _DN_T_RHS,
                              preferred_element_type=jnp.float32)
    sm = smp_ref[...]
    acc += jax.lax.dot_general(sm, wp_ref[...], _DN_T_RHS,
                               preferred_element_type=jnp.float32)
    acc += b_ref[...]
    mask_slot = w_ref.shape[1] - emb_dim
    e_mask = (jax.lax.broadcasted_iota(jnp.int32, (_PK, 1), 0) ==
              mask_slot).astype(jnp.float32)
    mcol = jnp.dot(sm, e_mask, preferred_element_type=jnp.float32)
    out_ref[...] = acc * mcol


def kernel(embeddings, visibility_scores, bbox_ltwh, keypoints_xyc, feats_masks, W, b):
    B, N = feats_masks.shape
    M = B * N
    emb_dim = embeddings.shape[-1]
    kp_dim = keypoints_xyc.shape[-2] * keypoints_xyc.shape[-1]
    token_dim = W.shape[0]
    n_small = kp_dim + 5

    smallP = jnp.concatenate(
        [visibility_scores.reshape(M, 1),
         bbox_ltwh.reshape(M, 4),
         keypoints_xyc.reshape(M, kp_dim),
         feats_masks.reshape(M, 1).astype(jnp.float32),
         jnp.zeros((M, _PK - n_small - 1), jnp.float32)],
        axis=1)
    w_pad = jnp.concatenate(
        [W[:, emb_dim:], jnp.zeros((token_dim, _PK - n_small), W.dtype)],
        axis=1)

    emb = embeddings.reshape(M, emb_dim)
    b2 = b.reshape(1, token_dim)

    grid = (M // _TM,)
    out = pl.pallas_call(
        _proj_body,
        grid=grid,
        in_specs=[
            pl.BlockSpec((_TM, emb_dim), lambda i: (i, 0)),
            pl.BlockSpec((_TM, _PK), lambda i: (i, 0)),
            pl.BlockSpec(W.shape, lambda i: (0, 0)),
            pl.BlockSpec((token_dim, _PK), lambda i: (0, 0)),
            pl.BlockSpec(b2.shape, lambda i: (0, 0)),
        ],
        out_specs=pl.BlockSpec((_TM, token_dim), lambda i: (i, 0)),
        out_shape=jax.ShapeDtypeStruct((M, token_dim), jnp.float32),
    )(emb, smallP, W, w_pad, b2)

    return out.reshape(B, N, token_dim)
